# Initial kernel scaffold; baseline (speedup 1.0000x reference)
import jax
import jax.numpy as jnp
from jax.experimental import pallas as pl


def kernel(logits, labels):
    raise NotImplementedError("write your pallas kernel here")


# speedup vs baseline: 31.6944x; 31.6944x over previous
import functools

import jax
import jax.numpy as jnp
from jax import lax
from jax.experimental import pallas as pl
from jax.experimental.pallas import tpu as pltpu
from jax.experimental.pallas import tpu_sc as plsc

N_TOTAL = 16 * 512 * 512
NW = 32
PER_W = N_TOTAL // NW
CHUNK = 8192
NCHUNK = PER_W // CHUNK
VPC = CHUNK // 16

NB = 8192
HI = 16.0
SCALE = NB / HI
NB2 = 2 * NB

_f32 = jnp.float32


def _sc_hist_body(logits_hbm, labels_hbm, hist_out, aux_out,
                  lbuf, ybuf, cnt, srelu, auxbuf):
    wid = lax.axis_index("s") * 2 + lax.axis_index("c")
    base = wid * PER_W

    zeros16 = jnp.zeros((16,), _f32)
    ones16 = jnp.ones((16,), _f32)

    def zero_body(i, _):
        cnt[pl.ds(i * 16, 16)] = zeros16
        srelu[pl.ds(i * 16, 16)] = zeros16
        return 0
    lax.fori_loop(0, NB2 // 16, zero_body, 0)

    def chunk_body(c, carry):
        sumlab, maxe = carry
        off = base + c * CHUNK
        pltpu.sync_copy(logits_hbm.at[pl.ds(off, CHUNK)], lbuf)
        pltpu.sync_copy(labels_hbm.at[pl.ds(off, CHUNK)], ybuf)

        def vec_body(j, inner):
            slab, mx = inner
            lg = lbuf[pl.ds(j * 16, 16)]
            y = ybuf[pl.ds(j * 16, 16)]
            e = 1.0 - lg * (2.0 * y - 1.0)
            r = jnp.maximum(e, 0.0)
            bf = jnp.minimum(jnp.maximum(e * SCALE, 0.0), float(NB - 1))
            bi = bf.astype(jnp.int32)
            idx2 = y.astype(jnp.int32) * NB + bi
            mask = bi >= 1
            plsc.addupdate_scatter(cnt, [idx2], ones16, mask=mask)
            plsc.addupdate_scatter(srelu, [idx2], r, mask=mask)
            return slab + y, jnp.maximum(mx, e)

        return lax.fori_loop(0, VPC, vec_body, (sumlab, maxe))

    init = (zeros16, jnp.full((16,), -jnp.inf, _f32))
    sumlab, maxe = lax.fori_loop(0, NCHUNK, chunk_body, init)

    auxbuf[pl.ds(0, 16)] = sumlab
    auxbuf[pl.ds(16, 16)] = maxe
    pltpu.sync_copy(cnt, hist_out.at[wid, 0])
    pltpu.sync_copy(srelu, hist_out.at[wid, 1])
    pltpu.sync_copy(auxbuf, aux_out.at[wid])


_sc_hist = pl.kernel(
    _sc_hist_body,
    out_type=(
        jax.ShapeDtypeStruct((NW, 2, NB2), _f32),
        jax.ShapeDtypeStruct((NW, 32), _f32),
    ),
    mesh=plsc.VectorSubcoreMesh(core_axis_name="c", subcore_axis_name="s"),
    scratch_types=[
        pltpu.VMEM((CHUNK,), _f32),
        pltpu.VMEM((CHUNK,), _f32),
        pltpu.VMEM((NB2,), _f32),
        pltpu.VMEM((NB2,), _f32),
        pltpu.VMEM((32,), _f32),
    ],
    compiler_params=pltpu.CompilerParams(needs_layout_passes=False),
)


def _tc_reduce_body(hist_ref, aux_ref, out_ref):
    h = hist_ref[...]
    hs = jnp.sum(h, axis=0)
    cnt = hs[0]
    srelu = hs[1]
    cnt_neg, cnt_pos = cnt[0:64], cnt[64:128]
    sr_neg, sr_pos = srelu[0:64], srelu[64:128]

    iota_r = lax.broadcasted_iota(jnp.int32, (128, 128), 0)
    iota_c = lax.broadcasted_iota(jnp.int32, (128, 128), 1)
    tri_incl = (iota_r <= iota_c).astype(_f32)
    ones_m = jnp.ones((128, 128), _f32)
    s_lo = (iota_r[0:64, 0:64] > iota_c[0:64, 0:64]).astype(_f32)

    def prefix_incl(a):
        row = jnp.dot(a, tri_incl, preferred_element_type=_f32,
                      precision=lax.Precision.HIGHEST)
        rs = jnp.dot(a, ones_m, preferred_element_type=_f32,
                     precision=lax.Precision.HIGHEST)
        off = jnp.dot(s_lo, rs, preferred_element_type=_f32,
                      precision=lax.Precision.HIGHEST)
        return row + off

    p_neg = prefix_incl(cnt_neg)
    p_pos = prefix_incl(cnt_pos)
    tot_neg = jnp.sum(cnt_neg)
    tot_pos = jnp.sum(cnt_pos)

    nba = tot_neg - p_neg
    pa = tot_pos - p_pos + cnt_pos
    t = cnt_neg

    g = jnp.sum(aux_ref[:, 0:16])
    maxe = jnp.max(aux_ref[:, 16:32])

    loss_pos = jnp.sum(sr_pos / (g + nba))
    den = (g + nba) * (g + nba + t)
    loss_neg = jnp.sum(jnp.where(t > 0.0, sr_neg * (g - pa) / den, 0.0))
    loss = jnp.where(g > 0.0, loss_pos + loss_neg, jnp.maximum(maxe, 0.0))
    out_ref[0, 0] = loss


_tc_reduce = pl.pallas_call(
    _tc_reduce_body,
    out_shape=jax.ShapeDtypeStruct((1, 1), _f32),
    out_specs=pl.BlockSpec(memory_space=pltpu.SMEM),
)


def kernel(logits, labels):
    lg = logits.reshape(-1)
    lb = labels.reshape(-1)
    hist, aux = _sc_hist(lg, lb)
    hist4 = hist.reshape(NW, 2, 128, 128)
    out = _tc_reduce(hist4, aux)
    return out[0, 0]


# speedup vs baseline: 68.2738x; 2.1541x over previous
import jax
import jax.numpy as jnp
from jax import lax
from jax.experimental import pallas as pl
from jax.experimental.pallas import tpu as pltpu
from jax.experimental.pallas import tpu_sc as plsc

N_TOTAL = 16 * 512 * 512
NW = 32
PER_W = N_TOTAL // NW
CHUNK = 8192
NCHUNK = PER_W // CHUNK
VPC = CHUNK // 16

NB = 8192
HI = 16.0
SCALE = NB / HI
NB2 = 2 * NB

_f32 = jnp.float32


def _sc_hist_body(logits_hbm, labels_hbm, hist_out, aux_out,
                  lbuf0, lbuf1, ybuf0, ybuf1, cnt, srelu, auxbuf, sem):
    lbufs = (lbuf0, lbuf1)
    ybufs = (ybuf0, ybuf1)
    wid = lax.axis_index("s") * 2 + lax.axis_index("c")
    base = wid * PER_W

    zeros16 = jnp.zeros((16,), _f32)
    ones16 = jnp.ones((16,), _f32)

    @plsc.parallel_loop(0, NB2 // 16, unroll=8)
    def _zero(i):
        cnt[pl.ds(i * 16, 16)] = zeros16
        srelu[pl.ds(i * 16, 16)] = zeros16

    def start(c):
        off = base + c * CHUNK
        slot = c % 2
        return (
            pltpu.async_copy(logits_hbm.at[pl.ds(off, CHUNK)],
                             lbufs[slot], sem.at[slot]),
            pltpu.async_copy(labels_hbm.at[pl.ds(off, CHUNK)],
                             ybufs[slot], sem.at[slot]),
        )

    pending = {0: start(0)}
    maxv = jnp.full((16,), -jnp.inf, _f32)
    for c in range(NCHUNK):
        if c + 1 < NCHUNK:
            pending[c + 1] = start(c + 1)
        for h in pending.pop(c):
            h.wait()
        lb_s = lbufs[c % 2]
        yb_s = ybufs[c % 2]

        def vec_body(j, mx, lb_s=lb_s, yb_s=yb_s):
            lg = lb_s[pl.ds(j * 16, 16)]
            y = yb_s[pl.ds(j * 16, 16)]
            lgs = lg * SCALE
            t = lgs * y
            es = (SCALE + lgs) - (t + t)
            rs = jnp.maximum(es, 0.0)
            b = jnp.minimum(rs, float(NB - 1))
            idx2 = (y * float(NB) + b).astype(jnp.int32)
            plsc.addupdate_scatter(cnt, [idx2], ones16)
            plsc.addupdate_scatter(srelu, [idx2], rs)
            return jnp.maximum(mx, es)

        maxv = plsc.parallel_loop(0, VPC, unroll=8, carry=maxv)(vec_body)

    auxbuf[pl.ds(0, 16)] = maxv
    pltpu.sync_copy(cnt, hist_out.at[wid, 0])
    pltpu.sync_copy(srelu, hist_out.at[wid, 1])
    pltpu.sync_copy(auxbuf, aux_out.at[wid])


_sc_hist = pl.kernel(
    _sc_hist_body,
    out_type=(
        jax.ShapeDtypeStruct((NW, 2, NB2), _f32),
        jax.ShapeDtypeStruct((NW, 16), _f32),
    ),
    mesh=plsc.VectorSubcoreMesh(core_axis_name="c", subcore_axis_name="s"),
    scratch_types=[
        pltpu.VMEM((CHUNK,), _f32),
        pltpu.VMEM((CHUNK,), _f32),
        pltpu.VMEM((CHUNK,), _f32),
        pltpu.VMEM((CHUNK,), _f32),
        pltpu.VMEM((NB2,), _f32),
        pltpu.VMEM((NB2,), _f32),
        pltpu.VMEM((16,), _f32),
        pltpu.SemaphoreType.DMA((2,)),
    ],
    compiler_params=pltpu.CompilerParams(needs_layout_passes=False),
)


def _tc_reduce_body(hist_ref, aux_ref, out_ref):
    h = hist_ref[...]
    hs = jnp.sum(h, axis=0)
    cnt = hs[0]
    srelu = hs[1]
    cnt_neg, cnt_pos = cnt[0:64], cnt[64:128]
    sr_neg, sr_pos = srelu[0:64], srelu[64:128]

    iota_r = lax.broadcasted_iota(jnp.int32, (128, 128), 0)
    iota_c = lax.broadcasted_iota(jnp.int32, (128, 128), 1)
    tri_incl = (iota_r <= iota_c).astype(_f32)
    ones_m = jnp.ones((128, 128), _f32)
    s_lo = (iota_r[0:64, 0:64] > iota_c[0:64, 0:64]).astype(_f32)

    def prefix_incl(a):
        row = jnp.dot(a, tri_incl, preferred_element_type=_f32,
                      precision=lax.Precision.HIGHEST)
        rs = jnp.dot(a, ones_m, preferred_element_type=_f32,
                     precision=lax.Precision.HIGHEST)
        off = jnp.dot(s_lo, rs, preferred_element_type=_f32,
                      precision=lax.Precision.HIGHEST)
        return row + off

    p_neg = prefix_incl(cnt_neg)
    p_pos = prefix_incl(cnt_pos)
    tot_neg = jnp.sum(cnt_neg)
    g = jnp.sum(cnt_pos)

    nba = tot_neg - p_neg
    pa = g - p_pos + cnt_pos
    t = cnt_neg

    maxe = jnp.max(aux_ref[...]) * (1.0 / SCALE)

    loss_pos = jnp.sum(sr_pos / (g + nba))
    den = (g + nba) * (g + nba + t)
    loss_neg = jnp.sum(jnp.where(t > 0.0, sr_neg * (g - pa) / den, 0.0))
    loss = (loss_pos + loss_neg) * (1.0 / SCALE)
    out_ref[0, 0] = jnp.where(g > 0.0, loss, jnp.maximum(maxe, 0.0))


_tc_reduce = pl.pallas_call(
    _tc_reduce_body,
    out_shape=jax.ShapeDtypeStruct((1, 1), _f32),
    out_specs=pl.BlockSpec(memory_space=pltpu.SMEM),
)


def kernel(logits, labels):
    lg = logits.reshape(-1)
    lb = labels.reshape(-1)
    hist, aux = _sc_hist(lg, lb)
    hist4 = hist.reshape(NW, 2, 128, 128)
    out = _tc_reduce(hist4, aux)
    return out[0, 0]


# speedup vs baseline: 74.7750x; 1.0952x over previous
import jax
import jax.numpy as jnp
from jax import lax
from jax.experimental import pallas as pl
from jax.experimental.pallas import tpu as pltpu
from jax.experimental.pallas import tpu_sc as plsc

N_TOTAL = 16 * 512 * 512
NW = 32
PER_W = N_TOTAL // NW
CHUNK = 8192
NCHUNK = PER_W // CHUNK
VPC = CHUNK // 16

NB = 1024
HI = 16.0
S16 = float(NB * 16 / HI)
Y16 = float(NB * 16)
CLAMP = float(NB * 16 - 1)
HW = 2 * NB * 16

_f32 = jnp.float32


def _sc_hist_body(logits_hbm, labels_hbm, hist_out, aux_out,
                  lbuf0, lbuf1, ybuf0, ybuf1, cnt, srelu, auxbuf, sem):
    lbufs = (lbuf0, lbuf1)
    ybufs = (ybuf0, ybuf1)
    wid = lax.axis_index("s") * 2 + lax.axis_index("c")
    base = wid * PER_W

    zeros16 = jnp.zeros((16,), _f32)
    ones16 = jnp.ones((16,), _f32)
    lane = lax.broadcasted_iota(jnp.int32, (16,), 0)

    @plsc.parallel_loop(0, HW // 16, unroll=8)
    def _zero(i):
        cnt[pl.ds(i * 16, 16)] = zeros16
        srelu[pl.ds(i * 16, 16)] = zeros16

    def start(c):
        off = base + c * CHUNK
        slot = c % 2
        return (
            pltpu.async_copy(logits_hbm.at[pl.ds(off, CHUNK)],
                             lbufs[slot], sem.at[slot]),
            pltpu.async_copy(labels_hbm.at[pl.ds(off, CHUNK)],
                             ybufs[slot], sem.at[slot]),
        )

    pending = {0: start(0)}
    maxv = jnp.full((16,), -jnp.inf, _f32)
    for c in range(NCHUNK):
        if c + 1 < NCHUNK:
            pending[c + 1] = start(c + 1)
        for h in pending.pop(c):
            h.wait()
        lb_s = lbufs[c % 2]
        yb_s = ybufs[c % 2]

        def vec_body(j, mx, lb_s=lb_s, yb_s=yb_s):
            lg = lb_s[pl.ds(j * 16, 16)]
            y = yb_s[pl.ds(j * 16, 16)]
            lgs = lg * S16
            t = lgs * y
            es = (S16 + lgs) - (t + t)
            rs = jnp.maximum(es, 0.0)
            b = jnp.minimum(rs, CLAMP)
            idxf = y * Y16 + b
            idx = (idxf.astype(jnp.int32) & -16) | lane
            plsc.addupdate_scatter(cnt, [idx], ones16)
            plsc.addupdate_scatter(srelu, [idx], rs)
            return jnp.maximum(mx, es)

        maxv = plsc.parallel_loop(0, VPC, unroll=8, carry=maxv)(vec_body)

    auxbuf[pl.ds(0, 16)] = maxv
    pltpu.sync_copy(cnt, hist_out.at[wid, 0])
    pltpu.sync_copy(srelu, hist_out.at[wid, 1])
    pltpu.sync_copy(auxbuf, aux_out.at[wid])


_sc_hist = pl.kernel(
    _sc_hist_body,
    out_type=(
        jax.ShapeDtypeStruct((NW, 2, HW), _f32),
        jax.ShapeDtypeStruct((NW, 16), _f32),
    ),
    mesh=plsc.VectorSubcoreMesh(core_axis_name="c", subcore_axis_name="s"),
    scratch_types=[
        pltpu.VMEM((CHUNK,), _f32),
        pltpu.VMEM((CHUNK,), _f32),
        pltpu.VMEM((CHUNK,), _f32),
        pltpu.VMEM((CHUNK,), _f32),
        pltpu.VMEM((HW,), _f32),
        pltpu.VMEM((HW,), _f32),
        pltpu.VMEM((16,), _f32),
        pltpu.SemaphoreType.DMA((2,)),
    ],
    compiler_params=pltpu.CompilerParams(needs_layout_passes=False),
)


def _tc_reduce_body(hist_ref, aux_ref, out_ref):
    h = hist_ref[...]
    hs = jnp.sum(h, axis=0)
    cn, cp = hs[0, 0:128], hs[0, 128:256]
    sn, sp = hs[1, 0:128], hs[1, 128:256]

    iota_r = lax.broadcasted_iota(jnp.int32, (128, 128), 0)
    iota_c = lax.broadcasted_iota(jnp.int32, (128, 128), 1)
    br = lax.shift_right_logical(iota_r, 4)
    bc = lax.shift_right_logical(iota_c, 4)
    m_incl = (br <= bc).astype(_f32)
    m_bin = (br == bc).astype(_f32)
    ones_m = jnp.ones((128, 128), _f32)
    s_lo = (iota_r > iota_c).astype(_f32)

    def mm(a, b):
        return jnp.dot(a, b, preferred_element_type=_f32,
                       precision=lax.Precision.HIGHEST)

    def fp(a):
        return mm(a, m_incl) + mm(s_lo, mm(a, ones_m))

    tot_n = jnp.sum(cn)
    g = jnp.sum(cp)

    nba = tot_n - fp(cn)
    pa = g - fp(cp) + mm(cp, m_bin)
    t = mm(cn, m_bin)

    maxe = jnp.max(aux_ref[...])

    loss_pos = jnp.sum(sp / (g + nba))
    den = (g + nba) * (g + nba + t)
    loss_neg = jnp.sum(jnp.where(t > 0.0, sn * (g - pa) / den, 0.0))
    loss = (loss_pos + loss_neg) * (1.0 / S16)
    out_ref[0, 0] = jnp.where(g > 0.0, loss, jnp.maximum(maxe, 0.0) * (1.0 / S16))


_tc_reduce = pl.pallas_call(
    _tc_reduce_body,
    out_shape=jax.ShapeDtypeStruct((1, 1), _f32),
    out_specs=pl.BlockSpec(memory_space=pltpu.SMEM),
)


def kernel(logits, labels):
    lg = logits.reshape(-1)
    lb = labels.reshape(-1)
    hist, aux = _sc_hist(lg, lb)
    hist4 = hist.reshape(NW, 2, 256, 128)
    out = _tc_reduce(hist4, aux)
    return out[0, 0]


# speedup vs baseline: 128.1432x; 1.7137x over previous
import jax
import jax.numpy as jnp
from jax import lax
from jax.experimental import pallas as pl
from jax.experimental.pallas import tpu as pltpu
from jax.experimental.pallas import tpu_sc as plsc

N_TOTAL = 16 * 512 * 512
NROW = N_TOTAL // 512
NW = 32
ROW_W = NROW // NW
RPC = 16
NCHUNK = ROW_W // RPC
VPC = RPC * 512 // 16

NB = 1024
HI = 16.0
S16 = float(NB * 16 / HI)
Y16 = float(NB * 16)
CLAMP = float(NB * 16 - 1)
HW = 2 * NB * 16
HR = HW // 128

_f32 = jnp.float32


def _sc_hist_body(logits_hbm, labels_hbm, hist_out, aux_out,
                  lbuf0, lbuf1, ybuf0, ybuf1, cnt, srelu, auxbuf, sem):
    lbufs = (lbuf0, lbuf1)
    ybufs = (ybuf0, ybuf1)
    wid = lax.axis_index("s") * 2 + lax.axis_index("c")
    base_row = wid * ROW_W

    zeros16 = jnp.zeros((16,), _f32)
    ones16 = jnp.ones((16,), _f32)
    lane = lax.broadcasted_iota(jnp.int32, (16,), 0)

    @plsc.parallel_loop(0, HW // 16, unroll=8)
    def _zero(i):
        r = lax.shift_right_logical(i, 3)
        c = lax.shift_left(i & 7, 4)
        cnt[r, pl.ds(c, 16)] = zeros16
        srelu[r, pl.ds(c, 16)] = zeros16

    def start(c):
        row0 = base_row + c * RPC
        slot = c % 2
        return (
            pltpu.async_copy(logits_hbm.at[pl.ds(row0, RPC), :],
                             lbufs[slot], sem.at[slot]),
            pltpu.async_copy(labels_hbm.at[pl.ds(row0, RPC), :],
                             ybufs[slot], sem.at[slot]),
        )

    pending = {0: start(0)}
    maxv = jnp.full((16,), -jnp.inf, _f32)
    for c in range(NCHUNK):
        if c + 1 < NCHUNK:
            pending[c + 1] = start(c + 1)
        for h in pending.pop(c):
            h.wait()
        lb_s = lbufs[c % 2]
        yb_s = ybufs[c % 2]

        def vec_body(j, mx, lb_s=lb_s, yb_s=yb_s):
            r = lax.shift_right_logical(j, 5)
            cc = lax.shift_left(j & 31, 4)
            lg = lb_s[r, pl.ds(cc, 16)]
            y = yb_s[r, pl.ds(cc, 16)]
            lgs = lg * S16
            t = lgs * y
            es = (S16 + lgs) - (t + t)
            rs = jnp.maximum(es, 0.0)
            b = jnp.minimum(rs, CLAMP)
            idxf = y * Y16 + b
            idx = (idxf.astype(jnp.int32) & -16) | lane
            ir = lax.shift_right_logical(idx, 7)
            ic = idx & 127
            plsc.addupdate_scatter(cnt, [ir, ic], ones16)
            plsc.addupdate_scatter(srelu, [ir, ic], rs)
            return jnp.maximum(mx, es)

        maxv = plsc.parallel_loop(0, VPC, unroll=8, carry=maxv)(vec_body)

    auxbuf[pl.ds(0, 16)] = maxv
    pltpu.sync_copy(cnt, hist_out.at[pl.ds(wid * 2 * HR, HR), :])
    pltpu.sync_copy(srelu, hist_out.at[pl.ds(wid * 2 * HR + HR, HR), :])
    pltpu.sync_copy(auxbuf, aux_out.at[pl.ds(wid * 16, 16)])


_sc_hist = pl.kernel(
    _sc_hist_body,
    out_type=(
        jax.ShapeDtypeStruct((NW * 2 * HR, 128), _f32),
        jax.ShapeDtypeStruct((NW * 16,), _f32),
    ),
    mesh=plsc.VectorSubcoreMesh(core_axis_name="c", subcore_axis_name="s"),
    scratch_types=[
        pltpu.VMEM((RPC, 512), _f32),
        pltpu.VMEM((RPC, 512), _f32),
        pltpu.VMEM((RPC, 512), _f32),
        pltpu.VMEM((RPC, 512), _f32),
        pltpu.VMEM((HR, 128), _f32),
        pltpu.VMEM((HR, 128), _f32),
        pltpu.VMEM((16,), _f32),
        pltpu.SemaphoreType.DMA((2,)),
    ],
    compiler_params=pltpu.CompilerParams(
        needs_layout_passes=False, use_tc_tiling_on_sc=True),
)


def _tc_reduce_body(hist_ref, aux_ref, out_ref):
    h = hist_ref[...]
    hs = jnp.sum(h, axis=0)
    cn, cp = hs[0, 0:128], hs[0, 128:256]
    sn, sp = hs[1, 0:128], hs[1, 128:256]

    iota_r = lax.broadcasted_iota(jnp.int32, (128, 128), 0)
    iota_c = lax.broadcasted_iota(jnp.int32, (128, 128), 1)
    br = lax.shift_right_logical(iota_r, 4)
    bc = lax.shift_right_logical(iota_c, 4)
    m_incl = (br <= bc).astype(_f32)
    m_bin = (br == bc).astype(_f32)
    ones_m = jnp.ones((128, 128), _f32)
    s_lo = (iota_r > iota_c).astype(_f32)

    def mm(a, b):
        return jnp.dot(a, b, preferred_element_type=_f32,
                       precision=lax.Precision.HIGHEST)

    def fp(a):
        return mm(a, m_incl) + mm(s_lo, mm(a, ones_m))

    tot_n = jnp.sum(cn)
    g = jnp.sum(cp)

    nba = tot_n - fp(cn)
    pa = g - fp(cp) + mm(cp, m_bin)
    t = mm(cn, m_bin)

    maxe = jnp.max(aux_ref[...])

    loss_pos = jnp.sum(sp / (g + nba))
    den = (g + nba) * (g + nba + t)
    loss_neg = jnp.sum(jnp.where(t > 0.0, sn * (g - pa) / den, 0.0))
    loss = (loss_pos + loss_neg) * (1.0 / S16)
    out_ref[0, 0] = jnp.where(g > 0.0, loss, jnp.maximum(maxe, 0.0) * (1.0 / S16))


_tc_reduce = pl.pallas_call(
    _tc_reduce_body,
    out_shape=jax.ShapeDtypeStruct((1, 1), _f32),
    out_specs=pl.BlockSpec(memory_space=pltpu.SMEM),
)


def kernel(logits, labels):
    lg = logits.reshape(NROW, 512)
    lb = labels.reshape(NROW, 512)
    hist, aux = _sc_hist(lg, lb)
    hist4 = hist.reshape(NW, 2, HR, 128)
    out = _tc_reduce(hist4, aux.reshape(4, 128))
    return out[0, 0]
